# baseline (speedup 1.0000x reference)
import functools
import math

import jax
import jax.numpy as jnp
from jax import lax
from jax.experimental import pallas as pl
from jax.experimental.pallas import tpu as pltpu
from jax.experimental.pallas import tpu_sc as plsc

N = 10000
E = 160000
FEAT = 256
RATIO = 0.8
K1 = math.ceil(RATIO * N)
K2 = math.ceil(RATIO * K1)
K3 = math.ceil(RATIO * K2)

NT = 32
CHUNK = 128
IPP = 2048
SLOTS = IPP + 2 * CHUNK
E_PAD = ((E + IPP - 1) // IPP) * IPP
NPH = E_PAD // IPP

_LANE = None


def _make_segsum(n):
    nn = ((n + NT - 1) // NT + 7) // 8 * 8
    nacc = nn + 1
    mesh = plsc.VectorSubcoreMesh(core_axis_name="c", subcore_axis_name="s")

    @functools.partial(
        pl.kernel,
        mesh=mesh,
        compiler_params=pltpu.CompilerParams(needs_layout_passes=False),
        out_type=jax.ShapeDtypeStruct((NT * nn, FEAT), jnp.float32),
        scratch_types=[
            pltpu.VMEM((IPP,), jnp.int32),
            pltpu.VMEM((IPP,), jnp.int32),
            pltpu.VMEM((SLOTS,), jnp.int32),
            pltpu.VMEM((SLOTS,), jnp.int32),
            pltpu.VMEM((CHUNK, FEAT), jnp.float32),
            pltpu.VMEM((nacc, FEAT), jnp.float32),
            pltpu.SemaphoreType.DMA,
        ],
    )
    def k(h_hbm, src_hbm, dst_hbm, out_hbm,
          src_v, dst_v, stg_src, stg_dst, rows_v, acc_v, sem):
        c = lax.axis_index("c")
        s = lax.axis_index("s")
        t = c * 16 + s
        base = t * nn

        lane = jnp.arange(16, dtype=jnp.int32)
        ones = jnp.ones((16,), jnp.int32)
        zrow = jnp.zeros((16,), jnp.float32)
        zidx = jnp.zeros((16,), jnp.int32)

        def zbody(r, _):
            rv = jnp.full((16,), r, jnp.int32)
            for fb in range(FEAT // 16):
                plsc.store_scatter(acc_v, [rv, fb * 16 + lane], zrow)
            return 0

        lax.fori_loop(0, nacc, zbody, 0)

        def chunk_accum(q):
            qb = q * CHUNK

            def edge_body(e, _):
                ev = jnp.full((16,), qb + e, jnp.int32)
                el = jnp.full((16,), e, jnp.int32)
                dloc = plsc.load_gather(stg_dst, [ev]) - base
                for fb in range(FEAT // 16):
                    fo = fb * 16 + lane
                    v = plsc.load_gather(rows_v, [el, fo])
                    plsc.addupdate_scatter(acc_v, [dloc, fo], v)
                return 0

            lax.fori_loop(0, CHUNK, edge_body, 0)

        def gather_chunk(q):
            pltpu.async_copy(
                h_hbm.at[stg_src.at[pl.ds(q * CHUNK, CHUNK)]], rows_v, sem
            ).wait()

        def scan_body(i, fill):
            sv = src_v[pl.ds(i * 16, 16)]
            dv = dst_v[pl.ds(i * 16, 16)]
            m = (dv >= base) & (dv < base + nn)
            pos = plsc.cumsum(ones, mask=m) + (fill - 1)
            plsc.store_scatter(stg_src, [pos], sv, mask=m)
            plsc.store_scatter(stg_dst, [pos], dv, mask=m)
            return fill + jnp.sum(jnp.where(m, 1, 0))

        def phase_body(ph, fill):
            pltpu.sync_copy(src_hbm.at[pl.ds(ph * IPP, IPP)], src_v)
            pltpu.sync_copy(dst_hbm.at[pl.ds(ph * IPP, IPP)], dst_v)
            fill = lax.fori_loop(0, IPP // 16, scan_body, fill)
            nch = fill // CHUNK

            def run_chunk(q, _):
                gather_chunk(q)
                chunk_accum(q)
                return 0

            lax.fori_loop(0, nch, run_chunk, 0)

            start = nch * CHUNK
            rem = fill - start
            for l2 in range(CHUNK // 16):
                pos = l2 * 16 + lane
                mr = pos < rem
                sv = plsc.load_gather(stg_src, [start + pos], mask=mr)
                dv = plsc.load_gather(stg_dst, [start + pos], mask=mr)
                plsc.store_scatter(stg_src, [pos], sv, mask=mr)
                plsc.store_scatter(stg_dst, [pos], dv, mask=mr)
            return rem

        fill = lax.fori_loop(0, NPH, phase_body, jnp.int32(0))

        for l2 in range(CHUNK // 16):
            pos = fill + l2 * 16 + lane
            plsc.store_scatter(stg_src, [pos], zidx)
            plsc.store_scatter(stg_dst, [pos],
                               jnp.full((16,), base + nn, jnp.int32))

        def run_tail(q, _):
            gather_chunk(q)
            chunk_accum(q)
            return 0

        lax.fori_loop(0, (fill + CHUNK - 1) // CHUNK, run_tail, 0)

        pltpu.sync_copy(acc_v.at[pl.ds(0, nn)], out_hbm.at[pl.ds(base, nn)])

    return k, nn


_segsum = {n: _make_segsum(n) for n in (N, K1, K2)}


def _pad_edges(src, dst):
    pad = E_PAD - E
    src_p = jnp.concatenate([src, jnp.zeros((pad,), jnp.int32)])
    dst_p = jnp.concatenate([dst, jnp.full((pad,), -1, jnp.int32)])
    return src_p, dst_p


def _graph_conv(h, src_p, dst_p, c, n):
    fn, nn = _segsum[n]
    agg = fn(h, src_p, dst_p)[:n]
    return agg @ c["W_rel"].T + c["b_rel"] + h @ c["W_root"].T


def _topk_pool(h, src_p, dst_p, p, k, n):
    score = jnp.tanh((h @ p) / (jnp.linalg.norm(p) + 1e-16))
    vals, perm = jax.lax.top_k(score, k)
    h_new = h[perm] * vals[:, None]
    newidx = jnp.full((n,), -1, dtype=jnp.int32).at[perm].set(
        jnp.arange(k, dtype=jnp.int32))
    valid = dst_p >= 0
    ns = newidx[src_p]
    nd = newidx[jnp.where(valid, dst_p, 0)]
    keep = valid & (ns >= 0) & (nd >= 0)
    ns = jnp.where(keep, ns, 0)
    nd = jnp.where(keep, nd, -1)
    return h_new, ns, nd


def _readout(h):
    return jnp.concatenate(
        [jnp.max(h, axis=0, keepdims=True), jnp.mean(h, axis=0, keepdims=True)],
        axis=1)


def _head_body(z_ref, w1_ref, b1_ref, w2_ref, b2_ref, w3_ref, b3_ref, o_ref):
    z = z_ref[...]
    z = jnp.maximum(jnp.dot(z, w1_ref[...].T, preferred_element_type=jnp.float32) + b1_ref[...], 0.0)
    z = jnp.maximum(jnp.dot(z, w2_ref[...].T, preferred_element_type=jnp.float32) + b2_ref[...], 0.0)
    logits = jnp.dot(z, w3_ref[...].T, preferred_element_type=jnp.float32) + b3_ref[...]
    m = jnp.max(logits, axis=0, keepdims=True)
    sh = logits - m
    o_ref[...] = sh - jnp.log(jnp.sum(jnp.exp(sh), axis=0, keepdims=True))


def kernel(params, x, edge_index, batch):
    src_p, dst_p = _pad_edges(edge_index[0], edge_index[1])
    h = params["emb"][x]
    h = jax.nn.relu(_graph_conv(h, src_p, dst_p, params["conv1"], N))
    h, src_p, dst_p = _topk_pool(h, src_p, dst_p, params["p1"], K1, N)
    r1 = _readout(h)
    h = jax.nn.relu(_graph_conv(h, src_p, dst_p, params["conv2"], K1))
    h, src_p, dst_p = _topk_pool(h, src_p, dst_p, params["p2"], K2, K1)
    r2 = _readout(h)
    h = jax.nn.relu(_graph_conv(h, src_p, dst_p, params["conv3"], K2))
    h, src_p, dst_p = _topk_pool(h, src_p, dst_p, params["p3"], K3, K2)
    r3 = _readout(h)
    z = r1 + r2 + r3

    out = pl.pallas_call(
        _head_body,
        out_shape=jax.ShapeDtypeStruct((1, 10), jnp.float32),
    )(
        z,
        params["lin1"]["W"], params["lin1"]["b"][None, :],
        params["lin2"]["W"], params["lin2"]["b"][None, :],
        params["lin3"]["W"], params["lin3"]["b"][None, :],
    )
    return out


# speedup vs baseline: 3.8549x; 3.8549x over previous
import functools
import math

import jax
import jax.numpy as jnp
from jax import lax
from jax.experimental import pallas as pl
from jax.experimental.pallas import tpu as pltpu
from jax.experimental.pallas import tpu_sc as plsc

N = 10000
E = 160000
FEAT = 256
RATIO = 0.8
K1 = math.ceil(RATIO * N)
K2 = math.ceil(RATIO * K1)
K3 = math.ceil(RATIO * K2)

NT = 32
CHUNK = 128
IPP = 2048
SLOTS = IPP + 2 * CHUNK
E_PAD = ((E + IPP - 1) // IPP) * IPP
NPH = E_PAD // IPP
NMAP = N + 16


def _make_segsum(n, with_map):
    nn = ((n + NT - 1) // NT + 7) // 8 * 8
    nacc = nn + 1
    mesh = plsc.VectorSubcoreMesh(core_axis_name="c", subcore_axis_name="s")

    scratch = [
        pltpu.VMEM((IPP,), jnp.int32),
        pltpu.VMEM((IPP,), jnp.int32),
        pltpu.VMEM((SLOTS,), jnp.int32),
        pltpu.VMEM((SLOTS,), jnp.int32),
        pltpu.VMEM((CHUNK, FEAT), jnp.float32),
        pltpu.VMEM((nacc, FEAT), jnp.float32),
        pltpu.SemaphoreType.DMA,
    ]
    if with_map:
        scratch.append(pltpu.VMEM((NMAP,), jnp.int32))

    @functools.partial(
        pl.kernel,
        mesh=mesh,
        compiler_params=pltpu.CompilerParams(needs_layout_passes=False),
        out_type=jax.ShapeDtypeStruct((NT * nn, FEAT), jnp.float32),
        scratch_types=scratch,
    )
    def k(h_hbm, src_hbm, dst_hbm, *rest):
        if with_map:
            (cmap_hbm, out_hbm,
             src_v, dst_v, stg_src, stg_dst, rows_v, acc_v, sem, cmap_v) = rest
        else:
            (out_hbm,
             src_v, dst_v, stg_src, stg_dst, rows_v, acc_v, sem) = rest
        c = lax.axis_index("c")
        s = lax.axis_index("s")
        t = c * 16 + s
        base = t * nn

        lane = jnp.arange(16, dtype=jnp.int32)
        ones = jnp.ones((16,), jnp.int32)
        zrow = jnp.zeros((16,), jnp.float32)
        zidx = jnp.zeros((16,), jnp.int32)

        if with_map:
            pltpu.sync_copy(cmap_hbm, cmap_v)

        def zbody(r, _):
            rv = jnp.full((16,), r, jnp.int32)
            for fb in range(FEAT // 16):
                plsc.store_scatter(acc_v, [rv, fb * 16 + lane], zrow)
            return 0

        lax.fori_loop(0, nacc, zbody, 0)

        def chunk_accum(q):
            qb = q * CHUNK

            def edge_body(e, _):
                ev = jnp.full((16,), qb + e, jnp.int32)
                el = jnp.full((16,), e, jnp.int32)
                dloc = plsc.load_gather(stg_dst, [ev]) - base
                for fb in range(FEAT // 16):
                    fo = fb * 16 + lane
                    v = plsc.load_gather(rows_v, [el, fo])
                    plsc.addupdate_scatter(acc_v, [dloc, fo], v)
                return 0

            lax.fori_loop(0, CHUNK, edge_body, 0)

        def gather_chunk(q):
            pltpu.async_copy(
                h_hbm.at[stg_src.at[pl.ds(q * CHUNK, CHUNK)]], rows_v, sem
            ).wait()

        def scan_body(i, fill):
            sv = src_v[pl.ds(i * 16, 16)]
            dv = dst_v[pl.ds(i * 16, 16)]
            if with_map:
                sv = plsc.load_gather(cmap_v, [sv])
                dv = plsc.load_gather(cmap_v, [dv])
                m = (dv >= base) & (dv < base + nn) & (sv >= 0)
            else:
                m = (dv >= base) & (dv < base + nn)
            pos = plsc.cumsum(ones, mask=m) + (fill - 1)
            plsc.store_scatter(stg_src, [pos], sv, mask=m)
            plsc.store_scatter(stg_dst, [pos], dv, mask=m)
            return fill + jnp.sum(jnp.where(m, 1, 0))

        def phase_body(ph, fill):
            pltpu.sync_copy(src_hbm.at[pl.ds(ph * IPP, IPP)], src_v)
            pltpu.sync_copy(dst_hbm.at[pl.ds(ph * IPP, IPP)], dst_v)
            fill = lax.fori_loop(0, IPP // 16, scan_body, fill)
            nch = fill // CHUNK

            def run_chunk(q, _):
                gather_chunk(q)
                chunk_accum(q)
                return 0

            lax.fori_loop(0, nch, run_chunk, 0)

            start = nch * CHUNK
            rem = fill - start
            for l2 in range(CHUNK // 16):
                pos = l2 * 16 + lane
                mr = pos < rem
                sv = plsc.load_gather(stg_src, [start + pos], mask=mr)
                dv = plsc.load_gather(stg_dst, [start + pos], mask=mr)
                plsc.store_scatter(stg_src, [pos], sv, mask=mr)
                plsc.store_scatter(stg_dst, [pos], dv, mask=mr)
            return rem

        fill = lax.fori_loop(0, NPH, phase_body, jnp.int32(0))

        for l2 in range(CHUNK // 16):
            pos = fill + l2 * 16 + lane
            plsc.store_scatter(stg_src, [pos], zidx)
            plsc.store_scatter(stg_dst, [pos],
                               jnp.full((16,), base + nn, jnp.int32))

        def run_tail(q, _):
            gather_chunk(q)
            chunk_accum(q)
            return 0

        lax.fori_loop(0, (fill + CHUNK - 1) // CHUNK, run_tail, 0)

        pltpu.sync_copy(acc_v.at[pl.ds(0, nn)], out_hbm.at[pl.ds(base, nn)])

    return k, nn


_segsum = {N: _make_segsum(N, False),
           K1: _make_segsum(K1, True),
           K2: _make_segsum(K2, True)}


def _pad_edges(src, dst):
    pad = E_PAD - E
    src_p = jnp.concatenate([src, jnp.zeros((pad,), jnp.int32)])
    dst1_p = jnp.concatenate([dst, jnp.full((pad,), -1, jnp.int32)])
    dstN_p = jnp.concatenate([dst, jnp.full((pad,), N, jnp.int32)])
    return src_p, dst1_p, dstN_p


def _graph_conv(h, src_p, dst_p, cmap, c, n):
    fn, nn = _segsum[n]
    if cmap is None:
        agg = fn(h, src_p, dst_p)[:n]
    else:
        agg = fn(h, src_p, dst_p, cmap)[:n]
    return agg @ c["W_rel"].T + c["b_rel"] + h @ c["W_root"].T


def _topk_pool(h, p, k, n):
    score = jnp.tanh((h @ p) / (jnp.linalg.norm(p) + 1e-16))
    vals, perm = jax.lax.top_k(score, k)
    h_new = h[perm] * vals[:, None]
    newidx = jnp.full((n,), -1, dtype=jnp.int32).at[perm].set(
        jnp.arange(k, dtype=jnp.int32))
    return h_new, newidx


def _compose(cmap, newidx):
    safe = jnp.where(cmap >= 0, cmap, 0)
    return jnp.where(cmap >= 0, newidx[safe], -1)


def _readout(h):
    return jnp.concatenate(
        [jnp.max(h, axis=0, keepdims=True), jnp.mean(h, axis=0, keepdims=True)],
        axis=1)


def _head_body(z_ref, w1_ref, b1_ref, w2_ref, b2_ref, w3_ref, b3_ref, o_ref):
    z = z_ref[...]
    z = jnp.maximum(jnp.dot(z, w1_ref[...].T, preferred_element_type=jnp.float32) + b1_ref[...], 0.0)
    z = jnp.maximum(jnp.dot(z, w2_ref[...].T, preferred_element_type=jnp.float32) + b2_ref[...], 0.0)
    logits = jnp.dot(z, w3_ref[...].T, preferred_element_type=jnp.float32) + b3_ref[...]
    m = jnp.max(logits, axis=0, keepdims=True)
    sh = logits - m
    o_ref[...] = sh - jnp.log(jnp.sum(jnp.exp(sh), axis=0, keepdims=True))


def kernel(params, x, edge_index, batch):
    src_p, dst1_p, dstN_p = _pad_edges(edge_index[0], edge_index[1])
    h = params["emb"][x]
    h = jax.nn.relu(_graph_conv(h, src_p, dst1_p, None, params["conv1"], N))
    h, newidx = _topk_pool(h, params["p1"], K1, N)
    r1 = _readout(h)
    cmap = jnp.concatenate(
        [newidx, jnp.full((NMAP - N,), -1, jnp.int32)])
    h = jax.nn.relu(_graph_conv(h, src_p, dstN_p, cmap, params["conv2"], K1))
    h, newidx = _topk_pool(h, params["p2"], K2, K1)
    r2 = _readout(h)
    cmap = _compose(cmap, newidx)
    h = jax.nn.relu(_graph_conv(h, src_p, dstN_p, cmap, params["conv3"], K2))
    h, newidx = _topk_pool(h, params["p3"], K3, K2)
    r3 = _readout(h)
    z = r1 + r2 + r3

    out = pl.pallas_call(
        _head_body,
        out_shape=jax.ShapeDtypeStruct((1, 10), jnp.float32),
    )(
        z,
        params["lin1"]["W"], params["lin1"]["b"][None, :],
        params["lin2"]["W"], params["lin2"]["b"][None, :],
        params["lin3"]["W"], params["lin3"]["b"][None, :],
    )
    return out
